# baseline (speedup 1.0000x reference)
import jax
import jax.numpy as jnp
from jax import lax
from jax.experimental import pallas as pl
from jax.experimental.pallas import tpu as pltpu
from jax.experimental.pallas import tpu_sc as plsc

NPTS = 100000
NW = 32
BPW = 3200
PTOT = NW * BPW
NPLANES = 24
NV = BPW // 16
GRES = 128
NSRC = 8
CH = BPW // NSRC
CHV = CH // 16


def _rsqrt_nr(x):
    xi = lax.bitcast_convert_type(x, jnp.int32)
    yi = jnp.int32(0x5F3759DF) - lax.shift_right_arithmetic(xi, 1)
    y = lax.bitcast_convert_type(yi, jnp.float32)
    for _ in range(3):
        y = y * (1.5 - 0.5 * x * y * y)
    return y


def _round_f32(x):
    big = jnp.float32(8388608.0)
    return (x + big) - big


def _sc_body(pts_hbm, gp0, gp1, gp2, gp3, gp4, gp5, gp6, gp7, par_hbm, out_hbm,
             pts_v, par_v, idx_v,
             ub0, ub1, ub2, ub3, ub4, ub5, ub6, ub7, acc_v, sem):
    gps = (gp0, gp1, gp2, gp3, gp4, gp5, gp6, gp7)
    ubs = (ub0, ub1, ub2, ub3, ub4, ub5, ub6, ub7)

    wid = lax.axis_index("s") * 2 + lax.axis_index("c")
    base = wid * BPW
    pltpu.sync_copy(pts_hbm.at[pl.ds(base, BPW)], pts_v.at[pl.ds(0, BPW)])
    pltpu.sync_copy(pts_hbm.at[pl.ds(PTOT + base, BPW)],
                    pts_v.at[pl.ds(BPW, BPW)])
    pltpu.sync_copy(pts_hbm.at[pl.ds(2 * PTOT + base, BPW)],
                    pts_v.at[pl.ds(2 * BPW, BPW)])
    pltpu.sync_copy(par_hbm, par_v)

    lanes = lax.iota(jnp.int32, 16)

    g1 = par_v[pl.ds(96, 16)]
    g2 = par_v[pl.ds(112, 16)]
    svec = jnp.float32(GRES - 1) / (g2 - g1)
    ovec = -g1 * svec
    sx = svec[0]
    sy = svec[1]
    sz = svec[2]
    ox = ovec[0]
    oy = ovec[1]
    oz = ovec[2]
    qsx = g1[3]
    qsy = g1[4]
    qsz = g1[5]
    qox = g2[3]
    qoy = g2[4]
    qoz = g2[5]
    hi = jnp.float32(GRES - 1)

    def plane_body(j, carry):
        pv = par_v[pl.ds(4 * j, 16)]
        nx = pv[0]
        ny = pv[1]
        nz = pv[2]
        dd = pv[3]

        def idx_body(i, c):
            px = pts_v[pl.ds(i * 16, 16)]
            py = pts_v[pl.ds(BPW + i * 16, 16)]
            pz = pts_v[pl.ds(2 * BPW + i * 16, 16)]
            proj = px * nx + py * ny + pz * nz + dd
            rx = px - 2.0 * proj * nx
            ry = py - 2.0 * proj * ny
            rz = pz - 2.0 * proj * nz
            fx = _round_f32(jnp.minimum(jnp.maximum(rx * sx + ox, 0.0), hi))
            fy = _round_f32(jnp.minimum(jnp.maximum(ry * sy + oy, 0.0), hi))
            fz = _round_f32(jnp.minimum(jnp.maximum(rz * sz + oz, 0.0), hi))
            ix = fx.astype(jnp.int32)
            iy = fy.astype(jnp.int32)
            iz = fz.astype(jnp.int32)
            idx_v[pl.ds(i * 16, 16)] = (ix * (GRES * GRES) + iy * GRES) + iz
            return c

        lax.fori_loop(0, NV, idx_body, 0)

        cps = [pltpu.async_copy(gps[cc].at[idx_v.at[pl.ds(cc * CH, CH)]],
                                ubs[cc], sem)
               for cc in range(NSRC)]
        for cp in cps:
            cp.wait()

        acc = jnp.zeros((16,), jnp.float32)
        for cc in range(NSRC):
            ub = ubs[cc]

            def dist_body(k, acc, cc=cc, ub=ub):
                i = cc * CHV + k
                px = pts_v[pl.ds(i * 16, 16)]
                py = pts_v[pl.ds(BPW + i * 16, 16)]
                pz = pts_v[pl.ds(2 * BPW + i * 16, 16)]
                proj = px * nx + py * ny + pz * nz + dd
                rx = px - 2.0 * proj * nx
                ry = py - 2.0 * proj * ny
                rz = pz - 2.0 * proj * nz
                u = ub[pl.ds(k * 16, 16)]
                if cc:
                    u = u ^ cc
                ux = u & 2047
                uy = lax.shift_right_logical(u, 11) & 2047
                uz = lax.shift_right_logical(u, 22)
                dx = rx - (ux.astype(jnp.float32) * qsx + qox)
                dy = ry - (uy.astype(jnp.float32) * qsy + qoy)
                dz = rz - (uz.astype(jnp.float32) * qsz + qoz)
                d2 = jnp.maximum(dx * dx + dy * dy + dz * dz, 1e-30)
                dist = d2 * _rsqrt_nr(d2)
                dist = jnp.where(base + i * 16 + lanes < NPTS, dist, 0.0)
                return acc + dist

            acc = lax.fori_loop(0, CHV, dist_body, acc)
        acc_v[pl.ds(j * 16, 16)] = acc
        return carry

    lax.fori_loop(0, NPLANES, plane_body, 0)
    pltpu.sync_copy(acc_v, out_hbm.at[wid])


def _tc_finalize(part_ref, pp_ref, out_ref):
    avg_sd = jnp.sum(part_ref[...]) * (1.0 / (NPTS * 8.0))

    pp = pp_ref[...]
    n = pp[:, 0:3]
    norm = jnp.maximum(jnp.sqrt(jnp.sum(n * n, axis=1, keepdims=True)), 1e-12)
    nn = n / norm
    g = lax.dot_general(nn, nn, (((1,), (1,)), ((), ())),
                        preferred_element_type=jnp.float32)
    r = lax.broadcasted_iota(jnp.int32, (NPLANES, NPLANES), 0)
    c = lax.broadcasted_iota(jnp.int32, (NPLANES, NPLANES), 1)
    a = jnp.where((r // 3) == (c // 3),
                  g - (r == c).astype(jnp.float32), 0.0)
    avg_r = jnp.sum(a * a) * (1.0 / 8.0)

    col = lax.broadcasted_iota(jnp.int32, (1, 128), 1)
    out_ref[...] = jnp.where(
        col == 0, avg_sd + 0.25 * avg_r,
        jnp.where(col == 1, avg_sd, jnp.where(col == 2, avg_r, 0.0)))


def kernel(pred_params, surface_points, closest_point_grid, grid_min, grid_max):
    pts = jnp.pad(surface_points, ((0, PTOT - NPTS), (0, 0)))
    pts_planar = pts.T.reshape(-1)

    gtab = closest_point_grid.reshape(-1, 3)
    tmin = jnp.min(gtab, axis=0)
    tmax = jnp.max(gtab, axis=0)
    nlev = jnp.array([2047.0, 2047.0, 1023.0], jnp.float32)
    qstep = jnp.maximum((tmax - tmin) / nlev, 1e-30)
    q = jnp.clip(jnp.round((gtab - tmin[None, :]) / qstep[None, :]),
                 0.0, nlev[None, :]).astype(jnp.int32)
    gpacked = q[:, 0] | (q[:, 1] << 11) | (q[:, 2] << 22)
    gcopies = [gpacked ^ k for k in range(NSRC)]

    params = jnp.concatenate([
        pred_params.reshape(-1).astype(jnp.float32),
        grid_min.astype(jnp.float32),
        qstep,
        jnp.zeros((10,), jnp.float32),
        grid_max.astype(jnp.float32),
        tmin,
        jnp.zeros((10,), jnp.float32),
    ])

    mesh = plsc.VectorSubcoreMesh(core_axis_name="c", subcore_axis_name="s")
    partials = pl.kernel(
        _sc_body,
        out_type=jax.ShapeDtypeStruct((NW, NPLANES * 16), jnp.float32),
        mesh=mesh,
        scratch_types=(
            [pltpu.VMEM((3 * BPW,), jnp.float32),
             pltpu.VMEM((128,), jnp.float32),
             pltpu.VMEM((BPW,), jnp.int32)]
            + [pltpu.VMEM((CH,), jnp.int32)] * NSRC
            + [pltpu.VMEM((NPLANES * 16,), jnp.float32),
               pltpu.SemaphoreType.DMA]
        ),
    )(pts_planar, *gcopies, params)

    out = pl.pallas_call(
        _tc_finalize,
        out_shape=jax.ShapeDtypeStruct((1, 128), jnp.float32),
    )(partials, pred_params.reshape(NPLANES, 4))

    return (out[0, 0], out[0, 1], out[0, 2])


# speedup vs baseline: 1.4307x; 1.4307x over previous
import jax
import jax.numpy as jnp
from jax import lax
from jax.experimental import pallas as pl
from jax.experimental.pallas import tpu as pltpu
from jax.experimental.pallas import tpu_sc as plsc

NPTS = 100000
NW = 32
BPW = 3584
PTOT = NW * BPW
NPLANES = 24
NV = BPW // 16
GRES = 128
NSRC = 16
CH = BPW // NSRC
CHV = CH // 16


def _rsqrt_nr(x):
    xi = lax.bitcast_convert_type(x, jnp.int32)
    yi = jnp.int32(0x5F3759DF) - lax.shift_right_arithmetic(xi, 1)
    y = lax.bitcast_convert_type(yi, jnp.float32)
    for _ in range(3):
        y = y * (1.5 - 0.5 * x * y * y)
    return y


def _round_f32(x):
    big = jnp.float32(8388608.0)
    return (x + big) - big


def _sc_body(pts_hbm, *rest):
    gps = rest[0:NSRC]
    par_hbm = rest[NSRC]
    out_hbm = rest[NSRC + 1]
    pts_v = rest[NSRC + 2]
    par_v = rest[NSRC + 3]
    idx_v = rest[NSRC + 4]
    ubs = rest[NSRC + 5:2 * NSRC + 5]
    acc_v = rest[2 * NSRC + 5]
    sem = rest[2 * NSRC + 6]

    wid = lax.axis_index("s") * 2 + lax.axis_index("c")
    base = wid * BPW
    pltpu.sync_copy(pts_hbm.at[pl.ds(base, BPW)], pts_v.at[pl.ds(0, BPW)])
    pltpu.sync_copy(pts_hbm.at[pl.ds(PTOT + base, BPW)],
                    pts_v.at[pl.ds(BPW, BPW)])
    pltpu.sync_copy(pts_hbm.at[pl.ds(2 * PTOT + base, BPW)],
                    pts_v.at[pl.ds(2 * BPW, BPW)])
    pltpu.sync_copy(par_hbm, par_v)

    lanes = lax.iota(jnp.int32, 16)

    g1 = par_v[pl.ds(96, 16)]
    g2 = par_v[pl.ds(112, 16)]
    svec = jnp.float32(GRES - 1) / (g2 - g1)
    ovec = -g1 * svec
    sx = svec[0]
    sy = svec[1]
    sz = svec[2]
    ox = ovec[0]
    oy = ovec[1]
    oz = ovec[2]
    qsx = g1[3]
    qsy = g1[4]
    qsz = g1[5]
    qox = g2[3]
    qoy = g2[4]
    qoz = g2[5]
    hi = jnp.float32(GRES - 1)

    def plane_body(j, carry):
        pv = par_v[pl.ds(4 * j, 16)]
        nx = pv[0]
        ny = pv[1]
        nz = pv[2]
        dd = pv[3]

        def idx_body(i, c):
            px = pts_v[pl.ds(i * 16, 16)]
            py = pts_v[pl.ds(BPW + i * 16, 16)]
            pz = pts_v[pl.ds(2 * BPW + i * 16, 16)]
            proj = px * nx + py * ny + pz * nz + dd
            rx = px - 2.0 * proj * nx
            ry = py - 2.0 * proj * ny
            rz = pz - 2.0 * proj * nz
            fx = _round_f32(jnp.minimum(jnp.maximum(rx * sx + ox, 0.0), hi))
            fy = _round_f32(jnp.minimum(jnp.maximum(ry * sy + oy, 0.0), hi))
            fz = _round_f32(jnp.minimum(jnp.maximum(rz * sz + oz, 0.0), hi))
            ix = fx.astype(jnp.int32)
            iy = fy.astype(jnp.int32)
            iz = fz.astype(jnp.int32)
            idx_v[pl.ds(i * 16, 16)] = (ix * (GRES * GRES) + iy * GRES) + iz
            return c

        lax.fori_loop(0, NV, idx_body, 0)

        cps = [pltpu.async_copy(gps[cc].at[idx_v.at[pl.ds(cc * CH, CH)]],
                                ubs[cc], sem)
               for cc in range(NSRC)]
        for cp in cps:
            cp.wait()

        acc = jnp.zeros((16,), jnp.float32)
        for cc in range(NSRC):
            ub = ubs[cc]

            def dist_body(k, acc, cc=cc, ub=ub):
                i = cc * CHV + k
                px = pts_v[pl.ds(i * 16, 16)]
                py = pts_v[pl.ds(BPW + i * 16, 16)]
                pz = pts_v[pl.ds(2 * BPW + i * 16, 16)]
                proj = px * nx + py * ny + pz * nz + dd
                rx = px - 2.0 * proj * nx
                ry = py - 2.0 * proj * ny
                rz = pz - 2.0 * proj * nz
                u = ub[pl.ds(k * 16, 16)]
                if cc:
                    u = u ^ cc
                ux = u & 2047
                uy = lax.shift_right_logical(u, 11) & 2047
                uz = lax.shift_right_logical(u, 22)
                dx = rx - (ux.astype(jnp.float32) * qsx + qox)
                dy = ry - (uy.astype(jnp.float32) * qsy + qoy)
                dz = rz - (uz.astype(jnp.float32) * qsz + qoz)
                d2 = jnp.maximum(dx * dx + dy * dy + dz * dz, 1e-30)
                dist = d2 * _rsqrt_nr(d2)
                dist = jnp.where(base + i * 16 + lanes < NPTS, dist, 0.0)
                return acc + dist

            acc = lax.fori_loop(0, CHV, dist_body, acc)
        acc_v[pl.ds(j * 16, 16)] = acc
        return carry

    lax.fori_loop(0, NPLANES, plane_body, 0)
    pltpu.sync_copy(acc_v, out_hbm.at[wid])


def _tc_finalize(part_ref, pp_ref, out_ref):
    avg_sd = jnp.sum(part_ref[...]) * (1.0 / (NPTS * 8.0))

    pp = pp_ref[...]
    n = pp[:, 0:3]
    norm = jnp.maximum(jnp.sqrt(jnp.sum(n * n, axis=1, keepdims=True)), 1e-12)
    nn = n / norm
    g = lax.dot_general(nn, nn, (((1,), (1,)), ((), ())),
                        preferred_element_type=jnp.float32)
    r = lax.broadcasted_iota(jnp.int32, (NPLANES, NPLANES), 0)
    c = lax.broadcasted_iota(jnp.int32, (NPLANES, NPLANES), 1)
    a = jnp.where((r // 3) == (c // 3),
                  g - (r == c).astype(jnp.float32), 0.0)
    avg_r = jnp.sum(a * a) * (1.0 / 8.0)

    col = lax.broadcasted_iota(jnp.int32, (1, 128), 1)
    out_ref[...] = jnp.where(
        col == 0, avg_sd + 0.25 * avg_r,
        jnp.where(col == 1, avg_sd, jnp.where(col == 2, avg_r, 0.0)))


def kernel(pred_params, surface_points, closest_point_grid, grid_min, grid_max):
    pts = jnp.pad(surface_points, ((0, PTOT - NPTS), (0, 0)))
    pts_planar = pts.T.reshape(-1)

    gtab = closest_point_grid.reshape(-1, 3)
    tmin = jnp.min(gtab, axis=0)
    tmax = jnp.max(gtab, axis=0)
    nlev = jnp.array([2047.0, 2047.0, 1023.0], jnp.float32)
    qstep = jnp.maximum((tmax - tmin) / nlev, 1e-30)
    q = jnp.clip(jnp.round((gtab - tmin[None, :]) / qstep[None, :]),
                 0.0, nlev[None, :]).astype(jnp.int32)
    gpacked = q[:, 0] | (q[:, 1] << 11) | (q[:, 2] << 22)
    gcopies = [gpacked ^ k for k in range(NSRC)]

    params = jnp.concatenate([
        pred_params.reshape(-1).astype(jnp.float32),
        grid_min.astype(jnp.float32),
        qstep,
        jnp.zeros((10,), jnp.float32),
        grid_max.astype(jnp.float32),
        tmin,
        jnp.zeros((10,), jnp.float32),
    ])

    mesh = plsc.VectorSubcoreMesh(core_axis_name="c", subcore_axis_name="s")
    partials = pl.kernel(
        _sc_body,
        out_type=jax.ShapeDtypeStruct((NW, NPLANES * 16), jnp.float32),
        mesh=mesh,
        scratch_types=(
            [pltpu.VMEM((3 * BPW,), jnp.float32),
             pltpu.VMEM((128,), jnp.float32),
             pltpu.VMEM((BPW,), jnp.int32)]
            + [pltpu.VMEM((CH,), jnp.int32)] * NSRC
            + [pltpu.VMEM((NPLANES * 16,), jnp.float32),
               pltpu.SemaphoreType.DMA]
        ),
    )(pts_planar, *gcopies, params)

    out = pl.pallas_call(
        _tc_finalize,
        out_shape=jax.ShapeDtypeStruct((1, 128), jnp.float32),
    )(partials, pred_params.reshape(NPLANES, 4))

    return (out[0, 0], out[0, 1], out[0, 2])


# speedup vs baseline: 1.8442x; 1.2891x over previous
import jax
import jax.numpy as jnp
from jax import lax
from jax.experimental import pallas as pl
from jax.experimental.pallas import tpu as pltpu
from jax.experimental.pallas import tpu_sc as plsc

NPTS = 100000
NW = 32
BPW = 3200
PTOT = NW * BPW
NPLANES = 24
NV = BPW // 16
GRES = 128
NSRC = 25
CH = BPW // NSRC
CHV = CH // 16


def _rsqrt_nr(x):
    xi = lax.bitcast_convert_type(x, jnp.int32)
    yi = jnp.int32(0x5F3759DF) - lax.shift_right_arithmetic(xi, 1)
    y = lax.bitcast_convert_type(yi, jnp.float32)
    for _ in range(3):
        y = y * (1.5 - 0.5 * x * y * y)
    return y


def _round_f32(x):
    big = jnp.float32(8388608.0)
    return (x + big) - big


def _sc_body(pts_hbm, *rest):
    gps = rest[0:NSRC]
    par_hbm = rest[NSRC]
    out_hbm = rest[NSRC + 1]
    pts_v = rest[NSRC + 2]
    par_v = rest[NSRC + 3]
    idx_v = rest[NSRC + 4]
    refl_v = rest[NSRC + 5]
    ubs = rest[NSRC + 6:2 * NSRC + 6]
    acc_v = rest[2 * NSRC + 6]
    sem = rest[2 * NSRC + 7]

    wid = lax.axis_index("s") * 2 + lax.axis_index("c")
    base = wid * BPW
    pltpu.sync_copy(pts_hbm.at[pl.ds(base, BPW)], pts_v.at[pl.ds(0, BPW)])
    pltpu.sync_copy(pts_hbm.at[pl.ds(PTOT + base, BPW)],
                    pts_v.at[pl.ds(BPW, BPW)])
    pltpu.sync_copy(pts_hbm.at[pl.ds(2 * PTOT + base, BPW)],
                    pts_v.at[pl.ds(2 * BPW, BPW)])
    pltpu.sync_copy(par_hbm, par_v)

    lanes = lax.iota(jnp.int32, 16)

    g1 = par_v[pl.ds(96, 16)]
    g2 = par_v[pl.ds(112, 16)]
    svec = jnp.float32(GRES - 1) / (g2 - g1)
    ovec = -g1 * svec
    sx = svec[0]
    sy = svec[1]
    sz = svec[2]
    ox = ovec[0]
    oy = ovec[1]
    oz = ovec[2]
    qsx = g1[3]
    qsy = g1[4]
    qsz = g1[5]
    qox = g2[3]
    qoy = g2[4]
    qoz = g2[5]
    hi = jnp.float32(GRES - 1)

    def plane_body(j, carry):
        pv = par_v[pl.ds(4 * j, 16)]
        nx = pv[0]
        ny = pv[1]
        nz = pv[2]
        dd = pv[3]

        def idx_body(i, c):
            px = pts_v[pl.ds(i * 16, 16)]
            py = pts_v[pl.ds(BPW + i * 16, 16)]
            pz = pts_v[pl.ds(2 * BPW + i * 16, 16)]
            proj = px * nx + py * ny + pz * nz + dd
            rx = px - 2.0 * proj * nx
            ry = py - 2.0 * proj * ny
            rz = pz - 2.0 * proj * nz
            refl_v[pl.ds(i * 16, 16)] = rx
            refl_v[pl.ds(BPW + i * 16, 16)] = ry
            refl_v[pl.ds(2 * BPW + i * 16, 16)] = rz
            fx = _round_f32(jnp.minimum(jnp.maximum(rx * sx + ox, 0.0), hi))
            fy = _round_f32(jnp.minimum(jnp.maximum(ry * sy + oy, 0.0), hi))
            fz = _round_f32(jnp.minimum(jnp.maximum(rz * sz + oz, 0.0), hi))
            ix = fx.astype(jnp.int32)
            iy = fy.astype(jnp.int32)
            iz = fz.astype(jnp.int32)
            idx_v[pl.ds(i * 16, 16)] = (ix * (GRES * GRES) + iy * GRES) + iz
            return c

        lax.fori_loop(0, NV, idx_body, 0)

        cps = [pltpu.async_copy(gps[cc].at[idx_v.at[pl.ds(cc * CH, CH)]],
                                ubs[cc], sem)
               for cc in range(NSRC)]
        for cp in cps:
            cp.wait()

        acc = jnp.zeros((16,), jnp.float32)
        for cc in range(NSRC):
            ub = ubs[cc]

            def dist_body(k, acc, cc=cc, ub=ub):
                i = cc * CHV + k
                rx = refl_v[pl.ds(i * 16, 16)]
                ry = refl_v[pl.ds(BPW + i * 16, 16)]
                rz = refl_v[pl.ds(2 * BPW + i * 16, 16)]
                u = ub[pl.ds(k * 16, 16)]
                if cc:
                    u = u ^ cc
                ux = u & 2047
                uy = lax.shift_right_logical(u, 11) & 2047
                uz = lax.shift_right_logical(u, 22)
                dx = rx - (ux.astype(jnp.float32) * qsx + qox)
                dy = ry - (uy.astype(jnp.float32) * qsy + qoy)
                dz = rz - (uz.astype(jnp.float32) * qsz + qoz)
                d2 = jnp.maximum(dx * dx + dy * dy + dz * dz, 1e-30)
                dist = d2 * _rsqrt_nr(d2)
                dist = jnp.where(base + i * 16 + lanes < NPTS, dist, 0.0)
                return acc + dist

            acc = lax.fori_loop(0, CHV, dist_body, acc)
        acc_v[pl.ds(j * 16, 16)] = acc
        return carry

    lax.fori_loop(0, NPLANES, plane_body, 0)
    pltpu.sync_copy(acc_v, out_hbm.at[wid])


def _tc_finalize(part_ref, pp_ref, out_ref):
    avg_sd = jnp.sum(part_ref[...]) * (1.0 / (NPTS * 8.0))

    pp = pp_ref[...]
    n = pp[:, 0:3]
    norm = jnp.maximum(jnp.sqrt(jnp.sum(n * n, axis=1, keepdims=True)), 1e-12)
    nn = n / norm
    g = lax.dot_general(nn, nn, (((1,), (1,)), ((), ())),
                        preferred_element_type=jnp.float32)
    r = lax.broadcasted_iota(jnp.int32, (NPLANES, NPLANES), 0)
    c = lax.broadcasted_iota(jnp.int32, (NPLANES, NPLANES), 1)
    a = jnp.where((r // 3) == (c // 3),
                  g - (r == c).astype(jnp.float32), 0.0)
    avg_r = jnp.sum(a * a) * (1.0 / 8.0)

    col = lax.broadcasted_iota(jnp.int32, (1, 128), 1)
    out_ref[...] = jnp.where(
        col == 0, avg_sd + 0.25 * avg_r,
        jnp.where(col == 1, avg_sd, jnp.where(col == 2, avg_r, 0.0)))


def kernel(pred_params, surface_points, closest_point_grid, grid_min, grid_max):
    pts = jnp.pad(surface_points, ((0, PTOT - NPTS), (0, 0)))
    pts_planar = pts.T.reshape(-1)

    gtab = closest_point_grid.reshape(-1, 3)
    tmin = jnp.min(gtab, axis=0)
    tmax = jnp.max(gtab, axis=0)
    nlev = jnp.array([2047.0, 2047.0, 1023.0], jnp.float32)
    qstep = jnp.maximum((tmax - tmin) / nlev, 1e-30)
    q = jnp.clip(jnp.round((gtab - tmin[None, :]) / qstep[None, :]),
                 0.0, nlev[None, :]).astype(jnp.int32)
    gpacked = q[:, 0] | (q[:, 1] << 11) | (q[:, 2] << 22)
    gcopies = [gpacked ^ k for k in range(NSRC)]

    params = jnp.concatenate([
        pred_params.reshape(-1).astype(jnp.float32),
        grid_min.astype(jnp.float32),
        qstep,
        jnp.zeros((10,), jnp.float32),
        grid_max.astype(jnp.float32),
        tmin,
        jnp.zeros((10,), jnp.float32),
    ])

    mesh = plsc.VectorSubcoreMesh(core_axis_name="c", subcore_axis_name="s")
    partials = pl.kernel(
        _sc_body,
        out_type=jax.ShapeDtypeStruct((NW, NPLANES * 16), jnp.float32),
        mesh=mesh,
        scratch_types=(
            [pltpu.VMEM((3 * BPW,), jnp.float32),
             pltpu.VMEM((128,), jnp.float32),
             pltpu.VMEM((BPW,), jnp.int32),
             pltpu.VMEM((3 * BPW,), jnp.float32)]
            + [pltpu.VMEM((CH,), jnp.int32)] * NSRC
            + [pltpu.VMEM((NPLANES * 16,), jnp.float32),
               pltpu.SemaphoreType.DMA]
        ),
    )(pts_planar, *gcopies, params)

    out = pl.pallas_call(
        _tc_finalize,
        out_shape=jax.ShapeDtypeStruct((1, 128), jnp.float32),
    )(partials, pred_params.reshape(NPLANES, 4))

    return (out[0, 0], out[0, 1], out[0, 2])
